# speedup vs baseline: 1.6727x; 1.6727x over previous
import jax
import jax.numpy as jnp
from jax.experimental import pallas as pl

SEQ_TILE = 512


def _add_pos_kernel(x_ref, pos_ref, o_ref):
    o_ref[...] = x_ref[...] + pos_ref[...]


def kernel(x, pos_table):
    batch, seq, embed = x.shape
    positions = pos_table[:seq]
    n_seq_tiles = seq // SEQ_TILE

    return pl.pallas_call(
        _add_pos_kernel,
        grid=(n_seq_tiles, batch),
        in_specs=[
            pl.BlockSpec((1, SEQ_TILE, embed), lambda i, j: (j, i, 0)),
            pl.BlockSpec((SEQ_TILE, embed), lambda i, j: (i, 0)),
        ],
        out_specs=pl.BlockSpec((1, SEQ_TILE, embed), lambda i, j: (j, i, 0)),
        out_shape=jax.ShapeDtypeStruct(x.shape, x.dtype),
    )(x, positions)


# speedup vs baseline: 1.8550x; 1.1090x over previous
import jax
import jax.numpy as jnp
from jax.experimental import pallas as pl

SEQ_TILE = 1024


def _add_pos_kernel(x_ref, pos_ref, o_ref):
    o_ref[...] = x_ref[...] + pos_ref[...]


def kernel(x, pos_table):
    batch, seq, embed = x.shape
    positions = pos_table[:seq]
    n_seq_tiles = seq // SEQ_TILE

    return pl.pallas_call(
        _add_pos_kernel,
        grid=(n_seq_tiles, batch),
        in_specs=[
            pl.BlockSpec((1, SEQ_TILE, embed), lambda i, j: (j, i, 0)),
            pl.BlockSpec((SEQ_TILE, embed), lambda i, j: (i, 0)),
        ],
        out_specs=pl.BlockSpec((1, SEQ_TILE, embed), lambda i, j: (j, i, 0)),
        out_shape=jax.ShapeDtypeStruct(x.shape, x.dtype),
    )(x, positions)


# speedup vs baseline: 1.9803x; 1.0675x over previous
import jax
import jax.numpy as jnp
from jax.experimental import pallas as pl

SEQ_TILE = 2048


def _add_pos_kernel(x_ref, pos_ref, o_ref):
    o_ref[...] = x_ref[...] + pos_ref[...]


def kernel(x, pos_table):
    batch, seq, embed = x.shape
    positions = pos_table[:seq]
    n_seq_tiles = seq // SEQ_TILE

    return pl.pallas_call(
        _add_pos_kernel,
        grid=(n_seq_tiles, batch),
        in_specs=[
            pl.BlockSpec((1, SEQ_TILE, embed), lambda i, j: (j, i, 0)),
            pl.BlockSpec((SEQ_TILE, embed), lambda i, j: (i, 0)),
        ],
        out_specs=pl.BlockSpec((1, SEQ_TILE, embed), lambda i, j: (j, i, 0)),
        out_shape=jax.ShapeDtypeStruct(x.shape, x.dtype),
    )(x, positions)
